# baseline (speedup 1.0000x reference)
import functools

import jax
import jax.numpy as jnp
from jax import lax
from jax.experimental import pallas as pl
from jax.experimental.pallas import tpu as pltpu
from jax.experimental.pallas import tpu_sc as plsc

_NC = 2
_NS = 16
_B = 128
_KB = 8
_CHUNK = _B * _KB
_TRASH = 128
_NBUF = 6
_F32 = jnp.float32


def _edge_aggregate_kernel(N, CH):
    ACC = N + _TRASH
    ZR = (-(-ACC // _NS) + 7) // 8 * 8
    WR = (-(-N // _NS) + 7) // 8 * 8
    mesh = plsc.VectorSubcoreMesh(core_axis_name="c", subcore_axis_name="s")

    @functools.partial(
        pl.kernel,
        out_type=(
            jax.ShapeDtypeStruct((N, 32), _F32),
            jax.ShapeDtypeStruct((N, 32), _F32),
        ),
        mesh=mesh,
        scratch_types=[
            pltpu.VMEM_SHARED((ACC, 32), _F32),
            pltpu.VMEM((_KB, _B), jnp.int32),
            pltpu.VMEM((_KB, _B), jnp.int32),
        ]
        + [pltpu.VMEM((_B, 32), _F32)] * _NBUF
        + [pltpu.SemaphoreType.DMA] * _NBUF,
        compiler_params=pltpu.CompilerParams(use_tc_tiling_on_sc=False),
    )
    def agg(t2, s2a, s2b, recv, zl, s0_out, s1_out, acc, sbuf, rbuf, *bs):
        bufs = bs[:_NBUF]
        gsems = bs[_NBUF:]
        c = lax.axis_index("c")
        s = lax.axis_index("s")
        z0 = jnp.minimum(s * ZR, ACC - ZR)
        w0 = jnp.minimum(s * WR, N - WR)
        pltpu.sync_copy(zl, acc.at[pl.ds(z0, ZR)])
        plsc.subcore_barrier()

        def run(s2_ref):
            def chunk(i, carry):
                r0 = (s * CH + i) * _KB
                pltpu.sync_copy(s2_ref.at[pl.ds(r0, _KB)], sbuf)
                pltpu.sync_copy(recv.at[pl.ds(r0, _KB)], rbuf)
                cps = [None] * _NBUF
                for j in range(_NBUF):
                    cps[j] = pltpu.async_copy(t2.at[sbuf.at[j]], bufs[j], gsems[j])
                for j in range(_KB):
                    b = j % _NBUF
                    cps[b].wait()
                    pltpu.sync_copy(bufs[b], acc.at[rbuf.at[j]], add=True)
                    if j + _NBUF < _KB:
                        cps[b] = pltpu.async_copy(
                            t2.at[sbuf.at[j + _NBUF]], bufs[b], gsems[b]
                        )
                return carry

            lax.fori_loop(0, CH, chunk, 0)

        pl.when(c == 0)(lambda: run(s2a))
        pl.when(c == 1)(lambda: run(s2b))
        plsc.subcore_barrier()
        pl.when(c == 0)(
            lambda: pltpu.sync_copy(acc.at[pl.ds(w0, WR)], s0_out.at[pl.ds(w0, WR)])
        )
        pl.when(c == 1)(
            lambda: pltpu.sync_copy(acc.at[pl.ds(w0, WR)], s1_out.at[pl.ds(w0, WR)])
        )

    return agg


def _edge_feature_aggregate_kernel(N, CHF):
    ZR = (-(-N // _NS) + 7) // 8 * 8
    mesh = plsc.VectorSubcoreMesh(core_axis_name="c", subcore_axis_name="s")

    @functools.partial(
        pl.kernel,
        out_type=(
            jax.ShapeDtypeStruct((N, 8), _F32),
            jax.ShapeDtypeStruct((N, 8), _F32),
        ),
        mesh=mesh,
        scratch_types=[
            pltpu.VMEM_SHARED((N, 8), _F32),
            pltpu.VMEM((_KB, _B), jnp.int32),
            pltpu.VMEM((_CHUNK, 8), _F32),
        ],
        compiler_params=pltpu.CompilerParams(use_tc_tiling_on_sc=False),
    )
    def fagg(ef8, recv, zf, f0_out, f1_out, acc, rbuf, erows):
        c = lax.axis_index("c")
        s = lax.axis_index("s")
        w = c * _NS + s
        z0 = jnp.minimum(s * ZR, N - ZR)
        pltpu.sync_copy(zf, acc.at[pl.ds(z0, ZR)])
        plsc.subcore_barrier()

        def chunk(i, carry):
            e0 = (w * CHF + i)
            pltpu.sync_copy(ef8.at[pl.ds(e0 * _CHUNK, _CHUNK)], erows)
            pltpu.sync_copy(recv.at[pl.ds(e0 * _KB, _KB)], rbuf)
            for j in range(_KB):
                pltpu.sync_copy(
                    erows.at[pl.ds(j * _B, _B)], acc.at[rbuf.at[j]], add=True
                )
            return carry

        lax.fori_loop(0, CHF, chunk, 0)
        plsc.subcore_barrier()
        pl.when(c == 0)(
            lambda: pltpu.sync_copy(acc.at[pl.ds(z0, ZR)], f0_out.at[pl.ds(z0, ZR)])
        )
        pl.when(c == 1)(
            lambda: pltpu.sync_copy(acc.at[pl.ds(z0, ZR)], f1_out.at[pl.ds(z0, ZR)])
        )

    return fagg


def _t0_call(N, BN, P, W_in, b_in2, wn2, wn64):
    def body(p, win, bin_, wn2_, wn64_, t0):
        h0 = jnp.dot(p[...], win[...], preferred_element_type=_F32) + bin_[...]
        t0[...] = (
            jnp.dot(h0, wn64_[...], preferred_element_type=_F32) + wn2_[...][0:1, :]
        )

    grid = (N // BN,)
    return pl.pallas_call(
        body,
        grid=grid,
        in_specs=[
            pl.BlockSpec((BN, 2), lambda i: (i, 0)),
            pl.BlockSpec((2, 64), lambda i: (0, 0)),
            pl.BlockSpec((1, 64), lambda i: (0, 0)),
            pl.BlockSpec((2, 64), lambda i: (0, 0)),
            pl.BlockSpec((64, 64), lambda i: (0, 0)),
        ],
        out_specs=pl.BlockSpec((BN, 64), lambda i: (i, 0)),
        out_shape=jax.ShapeDtypeStruct((N, 64), _F32),
    )(P, W_in, b_in2, wn2, wn64)


def _combine_call(N, BN, emit_t, S0, S1, F0, F1, V, wf, wout, bout, wn2=None, wn64=None):
    def body(s0, s1, f0, f1, v, wf_, wout_, bout_, *rest):
        h = jnp.concatenate([s0[...], s1[...]], axis=1)
        h = h + jnp.dot(f0[...] + f1[...], wf_[...], preferred_element_type=_F32)
        vn = v[...] + jnp.dot(h, wout_[...], preferred_element_type=_F32) + bout_[...]
        if emit_t:
            wn2_, wn64_, vn_ref, tn_ref = rest
            vn_ref[...] = vn
            tn_ref[...] = jnp.dot(vn, wn2_[...], preferred_element_type=_F32) + jnp.dot(
                h, wn64_[...], preferred_element_type=_F32
            )
        else:
            (vn_ref,) = rest
            vn_ref[...] = vn

    grid = (N // BN,)
    in_specs = [
        pl.BlockSpec((BN, 32), lambda i: (i, 0)),
        pl.BlockSpec((BN, 32), lambda i: (i, 0)),
        pl.BlockSpec((BN, 8), lambda i: (i, 0)),
        pl.BlockSpec((BN, 8), lambda i: (i, 0)),
        pl.BlockSpec((BN, 2), lambda i: (i, 0)),
        pl.BlockSpec((8, 64), lambda i: (0, 0)),
        pl.BlockSpec((64, 2), lambda i: (0, 0)),
        pl.BlockSpec((1, 2), lambda i: (0, 0)),
    ]
    args = [S0, S1, F0, F1, V, wf, wout, bout]
    if emit_t:
        in_specs += [
            pl.BlockSpec((2, 64), lambda i: (0, 0)),
            pl.BlockSpec((64, 64), lambda i: (0, 0)),
        ]
        args += [wn2, wn64]
        out_specs = (
            pl.BlockSpec((BN, 2), lambda i: (i, 0)),
            pl.BlockSpec((BN, 64), lambda i: (i, 0)),
        )
        out_shape = (
            jax.ShapeDtypeStruct((N, 2), _F32),
            jax.ShapeDtypeStruct((N, 64), _F32),
        )
    else:
        out_specs = pl.BlockSpec((BN, 2), lambda i: (i, 0))
        out_shape = jax.ShapeDtypeStruct((N, 2), _F32)
    return pl.pallas_call(
        body, grid=grid, in_specs=in_specs, out_specs=out_specs, out_shape=out_shape
    )(*args)


def kernel(P_Q_inj, senders, receivers, edge_features, W_in, b_in, W_msg, b_msg, W_out, b_out):
    N = P_Q_inj.shape[0]
    E = senders.shape[0]
    L = W_msg.shape[0]
    BN = 2000

    CH = -(-E // (_NS * _CHUNK))
    E_pad = CH * _NS * _CHUNK
    CHF = -(-E // (_NC * _NS * _CHUNK))
    E_padf = CHF * _NC * _NS * _CHUNK

    pad_l = E_pad - E
    spread_l = jnp.arange(pad_l, dtype=jnp.int32)
    sp = jnp.concatenate([senders, spread_l % N])
    s2a = (sp * 2).reshape(-1, _B)
    s2b = (sp * 2 + 1).reshape(-1, _B)
    rp = jnp.concatenate([receivers, N + (spread_l % _TRASH)]).reshape(-1, _B)

    pad_f = E_padf - E
    rf = jnp.concatenate(
        [receivers, jnp.arange(pad_f, dtype=jnp.int32) % N]
    ).reshape(-1, _B)
    ef8 = jnp.concatenate(
        [edge_features, jnp.ones((E, 1), _F32), jnp.zeros((E, 3), _F32)], axis=1
    )
    ef8 = jnp.concatenate([ef8, jnp.zeros((pad_f, 8), _F32)], axis=0)

    zl = jnp.zeros(((-(-(N + _TRASH) // _NS) + 7) // 8 * 8, 32), _F32)
    zf = jnp.zeros(((-(-N // _NS) + 7) // 8 * 8, 8), _F32)

    wn2 = W_msg[:, :2, :]
    wn64 = W_msg[:, 2:66, :]
    wf = jnp.concatenate(
        [W_msg[:, 66:70, :], b_msg[:, None, :], jnp.zeros((L, 3, 64), _F32)], axis=1
    )
    bout2 = b_out[:, None, :]
    bin2 = b_in[None, :]

    agg = _edge_aggregate_kernel(N, CH)
    fagg = _edge_feature_aggregate_kernel(N, CHF)

    F0, F1 = fagg(ef8, rf, zf)
    T = _t0_call(N, BN, P_Q_inj, W_in, bin2, wn2[0], wn64[0])
    V = jnp.concatenate([jnp.ones((N, 1), _F32), jnp.zeros((N, 1), _F32)], axis=1)

    for l in range(L):
        S0, S1 = agg(T.reshape(2 * N, 32), s2a, s2b, rp, zl)
        if l < L - 1:
            V, T = _combine_call(
                N, BN, True, S0, S1, F0, F1, V,
                wf[l], W_out[l], bout2[l], wn2[l + 1], wn64[l + 1],
            )
        else:
            V = _combine_call(
                N, BN, False, S0, S1, F0, F1, V, wf[l], W_out[l], bout2[l]
            )
    return V


# speedup vs baseline: 1.0359x; 1.0359x over previous
import functools

import jax
import jax.numpy as jnp
from jax import lax
from jax.experimental import pallas as pl
from jax.experimental.pallas import tpu as pltpu
from jax.experimental.pallas import tpu_sc as plsc

_NC = 2
_NS = 16
_B = 128
_KB = 12
_CHUNK = _B * _KB
_TRASH = 128
_NBUF = 6
_F32 = jnp.float32


def _edge_aggregate_kernel(N, CH):
    ACC = N + _TRASH
    ZR = (-(-ACC // _NS) + 7) // 8 * 8
    WR = (-(-N // _NS) + 7) // 8 * 8
    mesh = plsc.VectorSubcoreMesh(core_axis_name="c", subcore_axis_name="s")

    @functools.partial(
        pl.kernel,
        out_type=(
            jax.ShapeDtypeStruct((N, 32), _F32),
            jax.ShapeDtypeStruct((N, 32), _F32),
        ),
        mesh=mesh,
        scratch_types=[
            pltpu.VMEM_SHARED((ACC, 32), _F32),
            pltpu.VMEM((_KB, _B), jnp.int32),
            pltpu.VMEM((_KB, _B), jnp.int32),
        ]
        + [pltpu.VMEM((_B, 32), _F32)] * _NBUF
        + [pltpu.SemaphoreType.DMA] * _NBUF,
        compiler_params=pltpu.CompilerParams(use_tc_tiling_on_sc=False),
    )
    def agg(t2, s2a, s2b, recv, zl, s0_out, s1_out, acc, sbuf, rbuf, *bs):
        bufs = bs[:_NBUF]
        gsems = bs[_NBUF:]
        c = lax.axis_index("c")
        s = lax.axis_index("s")
        z0 = jnp.minimum(s * ZR, ACC - ZR)
        w0 = jnp.minimum(s * WR, N - WR)
        pltpu.sync_copy(zl, acc.at[pl.ds(z0, ZR)])
        plsc.subcore_barrier()

        def run(s2_ref):
            def chunk(i, carry):
                r0 = (s * CH + i) * _KB
                pltpu.sync_copy(s2_ref.at[pl.ds(r0, _KB)], sbuf)
                pltpu.sync_copy(recv.at[pl.ds(r0, _KB)], rbuf)
                cps = [None] * _NBUF
                for j in range(_NBUF):
                    cps[j] = pltpu.async_copy(t2.at[sbuf.at[j]], bufs[j], gsems[j])
                for j in range(_KB):
                    b = j % _NBUF
                    cps[b].wait()
                    pltpu.sync_copy(bufs[b], acc.at[rbuf.at[j]], add=True)
                    if j + _NBUF < _KB:
                        cps[b] = pltpu.async_copy(
                            t2.at[sbuf.at[j + _NBUF]], bufs[b], gsems[b]
                        )
                return carry

            lax.fori_loop(0, CH, chunk, 0)

        pl.when(c == 0)(lambda: run(s2a))
        pl.when(c == 1)(lambda: run(s2b))
        plsc.subcore_barrier()
        pl.when(c == 0)(
            lambda: pltpu.sync_copy(acc.at[pl.ds(w0, WR)], s0_out.at[pl.ds(w0, WR)])
        )
        pl.when(c == 1)(
            lambda: pltpu.sync_copy(acc.at[pl.ds(w0, WR)], s1_out.at[pl.ds(w0, WR)])
        )

    return agg


def _edge_feature_aggregate_kernel(N, CHF):
    ZR = (-(-N // _NS) + 7) // 8 * 8
    mesh = plsc.VectorSubcoreMesh(core_axis_name="c", subcore_axis_name="s")

    @functools.partial(
        pl.kernel,
        out_type=(
            jax.ShapeDtypeStruct((N, 8), _F32),
            jax.ShapeDtypeStruct((N, 8), _F32),
        ),
        mesh=mesh,
        scratch_types=[
            pltpu.VMEM_SHARED((N, 8), _F32),
            pltpu.VMEM((_KB, _B), jnp.int32),
            pltpu.VMEM((_CHUNK, 8), _F32),
        ],
        compiler_params=pltpu.CompilerParams(use_tc_tiling_on_sc=False),
    )
    def fagg(ef8, recv, zf, f0_out, f1_out, acc, rbuf, erows):
        c = lax.axis_index("c")
        s = lax.axis_index("s")
        w = c * _NS + s
        z0 = jnp.minimum(s * ZR, N - ZR)
        pltpu.sync_copy(zf, acc.at[pl.ds(z0, ZR)])
        plsc.subcore_barrier()

        def chunk(i, carry):
            e0 = (w * CHF + i)
            pltpu.sync_copy(ef8.at[pl.ds(e0 * _CHUNK, _CHUNK)], erows)
            pltpu.sync_copy(recv.at[pl.ds(e0 * _KB, _KB)], rbuf)
            for j in range(_KB):
                pltpu.sync_copy(
                    erows.at[pl.ds(j * _B, _B)], acc.at[rbuf.at[j]], add=True
                )
            return carry

        lax.fori_loop(0, CHF, chunk, 0)
        plsc.subcore_barrier()
        pl.when(c == 0)(
            lambda: pltpu.sync_copy(acc.at[pl.ds(z0, ZR)], f0_out.at[pl.ds(z0, ZR)])
        )
        pl.when(c == 1)(
            lambda: pltpu.sync_copy(acc.at[pl.ds(z0, ZR)], f1_out.at[pl.ds(z0, ZR)])
        )

    return fagg


def _t0_call(N, BN, P, W_in, b_in2, wn2, wn64):
    def body(p, win, bin_, wn2_, wn64_, t0):
        h0 = jnp.dot(p[...], win[...], preferred_element_type=_F32) + bin_[...]
        t0[...] = (
            jnp.dot(h0, wn64_[...], preferred_element_type=_F32) + wn2_[...][0:1, :]
        )

    grid = (N // BN,)
    return pl.pallas_call(
        body,
        grid=grid,
        in_specs=[
            pl.BlockSpec((BN, 2), lambda i: (i, 0)),
            pl.BlockSpec((2, 64), lambda i: (0, 0)),
            pl.BlockSpec((1, 64), lambda i: (0, 0)),
            pl.BlockSpec((2, 64), lambda i: (0, 0)),
            pl.BlockSpec((64, 64), lambda i: (0, 0)),
        ],
        out_specs=pl.BlockSpec((BN, 64), lambda i: (i, 0)),
        out_shape=jax.ShapeDtypeStruct((N, 64), _F32),
    )(P, W_in, b_in2, wn2, wn64)


def _combine_call(N, BN, emit_t, S0, S1, F0, F1, V, wf, wout, bout, wn2=None, wn64=None):
    def body(s0, s1, f0, f1, v, wf_, wout_, bout_, *rest):
        h = jnp.concatenate([s0[...], s1[...]], axis=1)
        h = h + jnp.dot(f0[...] + f1[...], wf_[...], preferred_element_type=_F32)
        vn = v[...] + jnp.dot(h, wout_[...], preferred_element_type=_F32) + bout_[...]
        if emit_t:
            wn2_, wn64_, vn_ref, tn_ref = rest
            vn_ref[...] = vn
            tn_ref[...] = jnp.dot(vn, wn2_[...], preferred_element_type=_F32) + jnp.dot(
                h, wn64_[...], preferred_element_type=_F32
            )
        else:
            (vn_ref,) = rest
            vn_ref[...] = vn

    grid = (N // BN,)
    in_specs = [
        pl.BlockSpec((BN, 32), lambda i: (i, 0)),
        pl.BlockSpec((BN, 32), lambda i: (i, 0)),
        pl.BlockSpec((BN, 8), lambda i: (i, 0)),
        pl.BlockSpec((BN, 8), lambda i: (i, 0)),
        pl.BlockSpec((BN, 2), lambda i: (i, 0)),
        pl.BlockSpec((8, 64), lambda i: (0, 0)),
        pl.BlockSpec((64, 2), lambda i: (0, 0)),
        pl.BlockSpec((1, 2), lambda i: (0, 0)),
    ]
    args = [S0, S1, F0, F1, V, wf, wout, bout]
    if emit_t:
        in_specs += [
            pl.BlockSpec((2, 64), lambda i: (0, 0)),
            pl.BlockSpec((64, 64), lambda i: (0, 0)),
        ]
        args += [wn2, wn64]
        out_specs = (
            pl.BlockSpec((BN, 2), lambda i: (i, 0)),
            pl.BlockSpec((BN, 64), lambda i: (i, 0)),
        )
        out_shape = (
            jax.ShapeDtypeStruct((N, 2), _F32),
            jax.ShapeDtypeStruct((N, 64), _F32),
        )
    else:
        out_specs = pl.BlockSpec((BN, 2), lambda i: (i, 0))
        out_shape = jax.ShapeDtypeStruct((N, 2), _F32)
    return pl.pallas_call(
        body, grid=grid, in_specs=in_specs, out_specs=out_specs, out_shape=out_shape
    )(*args)


def kernel(P_Q_inj, senders, receivers, edge_features, W_in, b_in, W_msg, b_msg, W_out, b_out):
    N = P_Q_inj.shape[0]
    E = senders.shape[0]
    L = W_msg.shape[0]
    BN = 2000

    CH = -(-E // (_NS * _CHUNK))
    E_pad = CH * _NS * _CHUNK
    CHF = -(-E // (_NC * _NS * _CHUNK))
    E_padf = CHF * _NC * _NS * _CHUNK

    pad_l = E_pad - E
    spread_l = jnp.arange(pad_l, dtype=jnp.int32)
    sp = jnp.concatenate([senders, spread_l % N])
    s2a = (sp * 2).reshape(-1, _B)
    s2b = (sp * 2 + 1).reshape(-1, _B)
    rp = jnp.concatenate([receivers, N + (spread_l % _TRASH)]).reshape(-1, _B)

    pad_f = E_padf - E
    rf = jnp.concatenate(
        [receivers, jnp.arange(pad_f, dtype=jnp.int32) % N]
    ).reshape(-1, _B)
    ef8 = jnp.concatenate(
        [edge_features, jnp.ones((E, 1), _F32), jnp.zeros((E, 3), _F32)], axis=1
    )
    ef8 = jnp.concatenate([ef8, jnp.zeros((pad_f, 8), _F32)], axis=0)

    zl = jnp.zeros(((-(-(N + _TRASH) // _NS) + 7) // 8 * 8, 32), _F32)
    zf = jnp.zeros(((-(-N // _NS) + 7) // 8 * 8, 8), _F32)

    wn2 = W_msg[:, :2, :]
    wn64 = W_msg[:, 2:66, :]
    wf = jnp.concatenate(
        [W_msg[:, 66:70, :], b_msg[:, None, :], jnp.zeros((L, 3, 64), _F32)], axis=1
    )
    bout2 = b_out[:, None, :]
    bin2 = b_in[None, :]

    agg = _edge_aggregate_kernel(N, CH)
    fagg = _edge_feature_aggregate_kernel(N, CHF)

    F0, F1 = fagg(ef8, rf, zf)
    T = _t0_call(N, BN, P_Q_inj, W_in, bin2, wn2[0], wn64[0])
    V = jnp.concatenate([jnp.ones((N, 1), _F32), jnp.zeros((N, 1), _F32)], axis=1)

    for l in range(L):
        S0, S1 = agg(T.reshape(2 * N, 32), s2a, s2b, rp, zl)
        if l < L - 1:
            V, T = _combine_call(
                N, BN, True, S0, S1, F0, F1, V,
                wf[l], W_out[l], bout2[l], wn2[l + 1], wn64[l + 1],
            )
        else:
            V = _combine_call(
                N, BN, False, S0, S1, F0, F1, V, wf[l], W_out[l], bout2[l]
            )
    return V
